# baseline (speedup 1.0000x reference)
import functools

import jax
import jax.numpy as jnp
from jax import lax
from jax.experimental import pallas as pl
from jax.experimental.pallas import tpu as pltpu
from jax.experimental.pallas import tpu_sc as plsc

B = 1024
CTX = 20
E = 64
V = 100000

NC, NS = 2, 16
NW = NC * NS
B_PER_W = B // NW
R_PER_W = B_PER_W * CTX
CHUNK = 128
NCHUNK = R_PER_W // CHUNK

LANE = 16
EV = E // LANE


EP = 64


def _pool_sc(x3, emb_table):
    mesh = plsc.VectorSubcoreMesh(core_axis_name="c", subcore_axis_name="s")

    @functools.partial(
        pl.kernel,
        out_type=jax.ShapeDtypeStruct((B, EP), jnp.float32),
        mesh=mesh,
        scratch_types=[
            pltpu.VMEM((NCHUNK, CHUNK), jnp.int32),
            pltpu.VMEM((R_PER_W, EP), jnp.float32),
            pltpu.VMEM((B_PER_W, EP), jnp.float32),
            pltpu.SemaphoreType.DMA,
        ],
        compiler_params=pltpu.CompilerParams(use_tc_tiling_on_sc=False),
    )
    def k(x_hbm, tbl_hbm, out_hbm, idx_v, rows_v, pooled_v, sem):
        wid = lax.axis_index("s") * NC + lax.axis_index("c")
        pltpu.sync_copy(x_hbm.at[wid], idx_v)
        copies = [
            pltpu.async_copy(
                tbl_hbm.at[idx_v.at[j]],
                rows_v.at[pl.ds(j * CHUNK, CHUNK)],
                sem,
            )
            for j in range(NCHUNK)
        ]
        for c in copies:
            c.wait()

        inv = jnp.float32(1.0 / CTX)

        def body_e(e, carry):
            def body_c(c, acc):
                r = e * CTX + c
                return tuple(
                    acc[v] + rows_v[r, pl.ds(v * LANE, LANE)] for v in range(EV)
                )

            acc = lax.fori_loop(
                0, CTX, body_c,
                tuple(jnp.zeros((LANE,), jnp.float32) for _ in range(EV)),
            )
            for v in range(EV):
                pooled_v[e, pl.ds(v * LANE, LANE)] = acc[v] * inv
            return carry

        lax.fori_loop(0, B_PER_W, body_e, 0)
        pltpu.sync_copy(pooled_v, out_hbm.at[pl.ds(wid * B_PER_W, B_PER_W)])

    return k(x3, emb_table)


NB = 2048
NSPLIT = 4
BS = B // NSPLIT


def _project_tc(embeds, W):
    dims = (((1,), (1,)), ((), ()))

    def mm(e_ref, w_ref, *o_refs):
        e = e_ref[...]
        w = w_ref[...]
        for k, o_ref in enumerate(o_refs):
            ek = lax.slice(e, (k * BS, 0), ((k + 1) * BS, E))
            o_ref[...] = lax.dot_general(
                ek, w, dims, preferred_element_type=jnp.float32,
            )

    return pl.pallas_call(
        mm,
        grid=(pl.cdiv(V, NB),),
        in_specs=[
            pl.BlockSpec((B, E), lambda j: (0, 0)),
            pl.BlockSpec((NB, E), lambda j: (j, 0)),
        ],
        out_specs=[
            pl.BlockSpec((BS, NB), lambda j: (0, j)) for _ in range(NSPLIT)
        ],
        out_shape=[
            jax.ShapeDtypeStruct((BS, V), jnp.float32) for _ in range(NSPLIT)
        ],
    )(embeds, W)


def kernel(x, emb_table, W):
    x3 = x.reshape(NW, NCHUNK, CHUNK)
    pooled = _pool_sc(x3, emb_table)
    embeds = pooled[:, :E]
    outs = _project_tc(embeds, W)
    return jnp.concatenate(outs, axis=0)


# speedup vs baseline: 1.3246x; 1.3246x over previous
import functools

import jax
import jax.numpy as jnp
from jax import lax
from jax.experimental import pallas as pl
from jax.experimental.pallas import tpu as pltpu
from jax.experimental.pallas import tpu_sc as plsc

B = 1024
CTX = 20
E = 64
V = 100000

NC, NS = 2, 16
NW = NC * NS
B_PER_W = B // NW
R_PER_W = B_PER_W * CTX
CHUNK = 128
NCHUNK = R_PER_W // CHUNK

LANE = 16
EV = E // LANE


EP = 64


def _pool_sc(x3, emb_table):
    mesh = plsc.VectorSubcoreMesh(core_axis_name="c", subcore_axis_name="s")

    @functools.partial(
        pl.kernel,
        out_type=jax.ShapeDtypeStruct((B, EP), jnp.float32),
        mesh=mesh,
        scratch_types=[
            pltpu.VMEM((NCHUNK, CHUNK), jnp.int32),
            pltpu.VMEM((R_PER_W, EP), jnp.float32),
            pltpu.VMEM((B_PER_W, EP), jnp.float32),
            pltpu.SemaphoreType.DMA,
        ],
        compiler_params=pltpu.CompilerParams(use_tc_tiling_on_sc=False),
    )
    def k(x_hbm, tbl_hbm, out_hbm, idx_v, rows_v, pooled_v, sem):
        wid = lax.axis_index("s") * NC + lax.axis_index("c")
        pltpu.sync_copy(x_hbm.at[wid], idx_v)
        copies = [
            pltpu.async_copy(
                tbl_hbm.at[idx_v.at[j]],
                rows_v.at[pl.ds(j * CHUNK, CHUNK)],
                sem,
            )
            for j in range(NCHUNK)
        ]
        for c in copies:
            c.wait()

        inv = jnp.float32(1.0 / CTX)

        def body_e(e, carry):
            def body_c(c, acc):
                r = e * CTX + c
                return tuple(
                    acc[v] + rows_v[r, pl.ds(v * LANE, LANE)] for v in range(EV)
                )

            acc = lax.fori_loop(
                0, CTX, body_c,
                tuple(jnp.zeros((LANE,), jnp.float32) for _ in range(EV)),
            )
            for v in range(EV):
                pooled_v[e, pl.ds(v * LANE, LANE)] = acc[v] * inv
            return carry

        lax.fori_loop(0, B_PER_W, body_e, 0)
        pltpu.sync_copy(pooled_v, out_hbm.at[pl.ds(wid * B_PER_W, B_PER_W)])

    return k(x3, emb_table)


NB = 2048
NSTEP = 49
EDGE = V - (NSTEP - 1) * NB
NBUF = 4


def _project_tc(embeds, W):
    def mm(e_ref, w_ref, o_ref, ob0, ob1, ob2, ob3, ebuf, s0, s1, s2, s3, se):
        j = pl.program_id(0)
        obufs = (ob0, ob1, ob2, ob3)
        sems = (s0, s1, s2, s3)
        dims = (((1,), (1,)), ((), ()))

        for k in range(NBUF):
            @pl.when(jnp.logical_and(lax.rem(j, NBUF) == k, j >= NBUF))
            def _(k=k):
                pltpu.make_async_copy(
                    obufs[k],
                    o_ref.at[:, pl.ds((j - NBUF) * NB, NB)],
                    sems[k],
                ).wait()

            @pl.when(jnp.logical_and(lax.rem(j, NBUF) == k, j < NSTEP - 1))
            def _(k=k):
                obufs[k][...] = lax.dot_general(
                    e_ref[...], w_ref[...], dims,
                    preferred_element_type=jnp.float32,
                )
                pltpu.make_async_copy(
                    obufs[k],
                    o_ref.at[:, pl.ds(j * NB, NB)],
                    sems[k],
                ).start(priority=k % 2)

        @pl.when(j == NSTEP - 1)
        def _():
            w_e = lax.slice(w_ref[...], (0, 0), (EDGE, E))
            ebuf[...] = lax.dot_general(
                e_ref[...], w_e, dims,
                preferred_element_type=jnp.float32,
            )
            pltpu.make_async_copy(
                ebuf,
                o_ref.at[:, pl.ds((NSTEP - 1) * NB, EDGE)],
                se,
            ).start()
            pltpu.make_async_copy(
                ebuf,
                o_ref.at[:, pl.ds((NSTEP - 1) * NB, EDGE)],
                se,
            ).wait()
            last_s = (NSTEP - 1) % NBUF
            for k in range(NBUF):
                if k == last_s:
                    continue
                jj = NSTEP - 1 - ((last_s - k) % NBUF)
                pltpu.make_async_copy(
                    obufs[k],
                    o_ref.at[:, pl.ds(jj * NB, NB)],
                    sems[k],
                ).wait()

    return pl.pallas_call(
        mm,
        grid=(NSTEP,),
        in_specs=[
            pl.BlockSpec((B, E), lambda j: (0, 0)),
            pl.BlockSpec((NB, E), lambda j: (j, 0)),
        ],
        out_specs=pl.BlockSpec(memory_space=pl.ANY),
        out_shape=jax.ShapeDtypeStruct((B, V), jnp.float32),
        scratch_shapes=[
            pltpu.VMEM((B, NB), jnp.float32),
            pltpu.VMEM((B, NB), jnp.float32),
            pltpu.VMEM((B, NB), jnp.float32),
            pltpu.VMEM((B, NB), jnp.float32),
            pltpu.VMEM((B, EDGE), jnp.float32),
            pltpu.SemaphoreType.DMA,
            pltpu.SemaphoreType.DMA,
            pltpu.SemaphoreType.DMA,
            pltpu.SemaphoreType.DMA,
            pltpu.SemaphoreType.DMA,
        ],
    )(embeds, W)


def kernel(x, emb_table, W):
    x3 = x.reshape(NW, NCHUNK, CHUNK)
    pooled = _pool_sc(x3, emb_table)
    embeds = pooled[:, :E]
    return _project_tc(embeds, W)


# speedup vs baseline: 1.3661x; 1.0314x over previous
import functools

import jax
import jax.numpy as jnp
from jax import lax
from jax.experimental import pallas as pl
from jax.experimental.pallas import tpu as pltpu
from jax.experimental.pallas import tpu_sc as plsc

B = 1024
CTX = 20
E = 64
V = 100000

NC, NS = 2, 16
NW = NC * NS
B_PER_W = B // NW
R_PER_W = B_PER_W * CTX
CHUNK = 128
NCHUNK = R_PER_W // CHUNK

LANE = 16
EV = E // LANE


EP = 64


def _pool_sc(x3, emb_table):
    mesh = plsc.VectorSubcoreMesh(core_axis_name="c", subcore_axis_name="s")

    @functools.partial(
        pl.kernel,
        out_type=jax.ShapeDtypeStruct((B, EP), jnp.float32),
        mesh=mesh,
        scratch_types=[
            pltpu.VMEM((NCHUNK, CHUNK), jnp.int32),
            pltpu.VMEM((R_PER_W, EP), jnp.float32),
            pltpu.VMEM((B_PER_W, EP), jnp.float32),
            pltpu.SemaphoreType.DMA,
        ],
        compiler_params=pltpu.CompilerParams(use_tc_tiling_on_sc=False),
    )
    def k(x_hbm, tbl_hbm, out_hbm, idx_v, rows_v, pooled_v, sem):
        wid = lax.axis_index("s") * NC + lax.axis_index("c")
        pltpu.sync_copy(x_hbm.at[wid], idx_v)
        copies = [
            pltpu.async_copy(
                tbl_hbm.at[idx_v.at[j]],
                rows_v.at[pl.ds(j * CHUNK, CHUNK)],
                sem,
            )
            for j in range(NCHUNK)
        ]
        for c in copies:
            c.wait()

        inv = jnp.float32(1.0 / CTX)

        def body_e(e, carry):
            def body_c(c, acc):
                r = e * CTX + c
                return tuple(
                    acc[v] + rows_v[r, pl.ds(v * LANE, LANE)] for v in range(EV)
                )

            acc = lax.fori_loop(
                0, CTX, body_c,
                tuple(jnp.zeros((LANE,), jnp.float32) for _ in range(EV)),
            )
            for v in range(EV):
                pooled_v[e, pl.ds(v * LANE, LANE)] = acc[v] * inv
            return carry

        lax.fori_loop(0, B_PER_W, body_e, 0)
        pltpu.sync_copy(pooled_v, out_hbm.at[pl.ds(wid * B_PER_W, B_PER_W)])

    return k(x3, emb_table)


BB = 16


def _project_tc(embeds, WT):
    def mm(e_ref, wt_ref, o_ref):
        o_ref[...] = lax.dot_general(
            e_ref[...], wt_ref[...],
            (((1,), (0,)), ((), ())),
            preferred_element_type=jnp.float32,
        )

    return pl.pallas_call(
        mm,
        grid=(B // BB,),
        in_specs=[
            pl.BlockSpec((BB, E), lambda j: (j, 0)),
            pl.BlockSpec((E, V), lambda j: (0, 0)),
        ],
        out_specs=pl.BlockSpec((BB, V), lambda j: (j, 0)),
        out_shape=jax.ShapeDtypeStruct((B, V), jnp.float32),
    )(embeds, WT)


def kernel(x, emb_table, W):
    x3 = x.reshape(NW, NCHUNK, CHUNK)
    pooled = _pool_sc(x3, emb_table)
    embeds = pooled[:, :E]
    WT = jnp.swapaxes(W, 0, 1)
    return _project_tc(embeds, WT)
